# baseline (speedup 1.0000x reference)
import functools
import numpy as np
import jax
import jax.numpy as jnp
from jax import lax
from jax.experimental import pallas as pl
from jax.experimental.pallas import tpu as pltpu
from jax.experimental.pallas import tpu_sc as plsc

_N = 10000
_E = 320000
_H = 128
_NLAYERS = 4
_FREQ = 256

_WORKERS = 32
_EH = _E // 2
_HB = _EH // 128
_BPW = _HB // _WORKERS
_EXTRA = _HB - _BPW * _WORKERS

_NBLK = 1000
_EBLK = 16000

_f32 = jnp.float32
_bf16 = jnp.bfloat16


def _silu(v):
    return v * jax.nn.sigmoid(v)


def _init_body(tf_ref, wt1, bt1, wt2, bt2, h_ref, wemb, bemb, we1a, we1b, be1,
               hh_o, prow_o, pcol_o, temb_o):
    te = _silu(tf_ref[...] @ wt1[...] + bt1[...]) @ wt2[...] + bt2[...]
    temb_o[...] = te
    hh = h_ref[...] @ wemb[...] + bemb[...] + te[0:1, :]
    hh_o[...] = hh
    prow_o[...] = hh @ we1a[...] + be1[...]
    pcol_o[...] = hh @ we1b[...]


def _edge_body(grow, xrp, xcp, eap, smat, rmat, w1r, w64, we2, be2,
               wc1, bc1, wc2, ef_o, tr_o):
    ep = _EBLK // 16
    dp = xrp[...] - xcp[...]
    radial_p = (dp * dp) @ smat[...]
    inv_p = 1.0 / (jnp.sqrt(radial_p + 1e-8) + 1.0)
    mask = (lax.broadcasted_iota(jnp.int32, (_EBLK, 16), 0) % 16
            == lax.broadcasted_iota(jnp.int32, (_EBLK, 16), 1)).astype(_f32)
    rad_x = jax.lax.broadcast_in_dim(radial_p, (ep, 16, 16), (0, 2))
    radial = jnp.sum(rad_x.reshape(_EBLK, 16) * mask, axis=1, keepdims=True)
    ea_x = jax.lax.broadcast_in_dim(eap[...], (ep, 16, 64), (0, 2))
    mask4 = (lax.broadcasted_iota(jnp.int32, (_EBLK, 64), 0) % 16
             == lax.broadcasted_iota(jnp.int32, (_EBLK, 64), 1) // 4).astype(_f32)
    ea_term = (ea_x.reshape(_EBLK, 64) * mask4) @ w64[...]
    m = _silu(grow[...] + radial * w1r[...] + ea_term)
    ef = _silu(m @ we2[...] + be2[...])
    cm = _silu(ef @ wc1[...] + bc1[...])
    s = jnp.sum(cm * wc2[...], axis=1, keepdims=True)
    s_p = jnp.sum((s * mask).reshape(ep, 16, 16), axis=1)
    ef_o[...] = ef
    tr_o[...] = dp * ((inv_p * s_p) @ rmat[...])


def _node_body(hh_ref, ph0, ph1, ph2, ph3, px0, px1, px2, px3,
               coord_ref, temb_ref,
               wn1a, wn1b, bn1, wn2, bn2, we1a, we1b, be1,
               hh_o, coord_o, prow_o, pcol_o):
    hh = hh_ref[...]
    aggh = (ph0[...] + ph1[...]) + (ph2[...] + ph3[...])
    o = _silu(hh @ wn1a[...] + aggh @ wn1b[...] + bn1[...]) @ wn2[...] + bn2[...]
    hhn = hh + o + temb_ref[0:1, :]
    hh_o[...] = hhn
    coord_o[...] = coord_ref[...] + (px0[...] + px1[...]) + (px2[...] + px3[...])
    prow_o[...] = hhn @ we1a[...] + be1[...]
    pcol_o[...] = hhn @ we1b[...]


def _final_body(hh_ref, ph0, ph1, ph2, ph3, px0, px1, px2, px3, coord_ref,
                wn1a, wn1b, bn1, wn2, bn2, wout, bout,
                hout_o, coord_o):
    hh = hh_ref[...]
    aggh = (ph0[...] + ph1[...]) + (ph2[...] + ph3[...])
    o = _silu(hh @ wn1a[...] + aggh @ wn1b[...] + bn1[...]) @ wn2[...] + bn2[...]
    hhn = hh + o
    hout_o[...] = hhn @ wout[...] + bout[...]
    coord_o[...] = coord_ref[...] + (px0[...] + px1[...]) + (px2[...] + px3[...])


_MESH = plsc.VectorSubcoreMesh(core_axis_name="c", subcore_axis_name="s")


def _gather_body(prow, pcol, xp, row1, col1,
                 grow_o, xr_o, xc_o,
                 idx_v, g_v, x_v, sems, sh_x):
    cid = lax.axis_index("c")
    sid = lax.axis_index("s")
    wid = sid * 2 + cid

    @pl.when(sid == 0)
    def _():
        pltpu.sync_copy(xp, sh_x)
    plsc.subcore_barrier()

    def start(b, s):
        base = b * 128
        pltpu.sync_copy(row1.at[pl.ds(base, 128)], idx_v[2 * s])
        pltpu.sync_copy(col1.at[pl.ds(base, 128)], idx_v[2 * s + 1])
        pltpu.async_copy(prow.at[idx_v[2 * s]], g_v[2 * s], sems[4 * s])
        pltpu.async_copy(pcol.at[idx_v[2 * s + 1]], g_v[2 * s + 1], sems[4 * s + 1])
        pltpu.async_copy(sh_x.at[idx_v[2 * s]], x_v[2 * s], sems[4 * s + 2])
        pltpu.async_copy(sh_x.at[idx_v[2 * s + 1]], x_v[2 * s + 1], sems[4 * s + 3])

    def finish(b, s):
        base = b * 128
        pltpu.make_async_copy(prow.at[idx_v[2 * s]], g_v[2 * s], sems[4 * s]).wait()
        pltpu.make_async_copy(pcol.at[idx_v[2 * s + 1]], g_v[2 * s + 1], sems[4 * s + 1]).wait()
        pltpu.make_async_copy(sh_x.at[idx_v[2 * s]], x_v[2 * s], sems[4 * s + 2]).wait()
        pltpu.make_async_copy(sh_x.at[idx_v[2 * s + 1]], x_v[2 * s + 1], sems[4 * s + 3]).wait()
        ga = g_v[2 * s]
        gb = g_v[2 * s + 1]

        def addrow(r, carry):
            for c in range(8):
                ga[r, pl.ds(c * 16, 16)] = (ga[r, pl.ds(c * 16, 16)]
                                            + gb[r, pl.ds(c * 16, 16)])
            return carry
        lax.fori_loop(0, 128, addrow, 0)
        pltpu.sync_copy(ga, grow_o.at[pl.ds(base, 128)])
        pltpu.sync_copy(x_v[2 * s], xr_o.at[pl.ds(base, 128)])
        pltpu.sync_copy(x_v[2 * s + 1], xc_o.at[pl.ds(base, 128)])

    base_b = wid * _BPW
    start(base_b, 0)

    def loop(jj, carry):
        b = base_b + 2 * jj
        start(b + 1, 1)
        finish(b, 0)
        start(b + 2, 0)
        finish(b + 1, 1)
        return carry
    lax.fori_loop(0, (_BPW - 1) // 2, loop, 0)
    finish(base_b + _BPW - 1, 0)

    @pl.when(wid < _EXTRA)
    def _():
        start(_WORKERS * _BPW + wid, 1)
        finish(_WORKERS * _BPW + wid, 1)


def _scatter_body(ef, tr, row1, zh, zx, ph_o, px_o,
                  idxb, ef_v, tr_v, sems, sh, sx):
    cid = lax.axis_index("c")
    sid = lax.axis_index("s")
    wid = sid * 2 + cid

    @pl.when(sid == 0)
    def _():
        pltpu.sync_copy(zh, sh)
        pltpu.sync_copy(zx, sx)
    plsc.subcore_barrier()

    def start(b, s):
        base = b * 128
        pltpu.async_copy(row1.at[pl.ds(base, 128)], idxb[s], sems[3 * s])
        pltpu.async_copy(ef.at[pl.ds(base, 128)], ef_v[s], sems[3 * s + 1])
        pltpu.async_copy(tr.at[pl.ds(base, 128)], tr_v[s], sems[3 * s + 2])

    def finish(b, s):
        base = b * 128
        pltpu.make_async_copy(row1.at[pl.ds(base, 128)], idxb[s], sems[3 * s]).wait()
        pltpu.make_async_copy(ef.at[pl.ds(base, 128)], ef_v[s], sems[3 * s + 1]).wait()
        pltpu.make_async_copy(tr.at[pl.ds(base, 128)], tr_v[s], sems[3 * s + 2]).wait()
        pltpu.sync_copy(ef_v[s], sh.at[idxb[s]], add=True)
        pltpu.sync_copy(tr_v[s], sx.at[idxb[s]], add=True)

    base_b = wid * _BPW
    start(base_b, 0)

    def loop(jj, carry):
        b = base_b + 2 * jj
        start(b + 1, 1)
        finish(b, 0)
        start(b + 2, 0)
        finish(b + 1, 1)
        return carry
    lax.fori_loop(0, (_BPW - 1) // 2, loop, 0)
    finish(base_b + _BPW - 1, 0)

    @pl.when(wid < _EXTRA)
    def _():
        start(_WORKERS * _BPW + wid, 1)
        finish(_WORKERS * _BPW + wid, 1)

    plsc.subcore_barrier()

    @pl.when(sid == 0)
    def _():
        pltpu.sync_copy(sh, ph_o.at[cid])
        pltpu.sync_copy(sx, px_o.at[cid])


def _make_sc_gather():
    return pl.kernel(
        _gather_body,
        out_type=(
            jax.ShapeDtypeStruct((_EH, _H), _f32),
            jax.ShapeDtypeStruct((_EH, 8), _f32),
            jax.ShapeDtypeStruct((_EH, 8), _f32),
        ),
        mesh=_MESH,
        scratch_types=[
            [pltpu.VMEM((128,), jnp.int32) for _ in range(4)],
            [pltpu.VMEM((128, _H), _f32) for _ in range(4)],
            [pltpu.VMEM((128, 8), _f32) for _ in range(4)],
            [pltpu.SemaphoreType.DMA for _ in range(8)],
            pltpu.VMEM_SHARED((_N, 8), _f32),
        ],
        compiler_params=pltpu.CompilerParams(use_tc_tiling_on_sc=False),
    )


def _make_sc_scatter():
    return pl.kernel(
        _scatter_body,
        out_type=(
            jax.ShapeDtypeStruct((2, _N, _H), _f32),
            jax.ShapeDtypeStruct((2, _N, 8), _f32),
        ),
        mesh=_MESH,
        scratch_types=[
            [pltpu.VMEM((128,), jnp.int32) for _ in range(2)],
            [pltpu.VMEM((128, _H), _f32) for _ in range(2)],
            [pltpu.VMEM((128, 8), _f32) for _ in range(2)],
            [pltpu.SemaphoreType.DMA for _ in range(6)],
            pltpu.VMEM_SHARED((_N, _H), _f32),
            pltpu.VMEM_SHARED((_N, 8), _f32),
        ],
        compiler_params=pltpu.CompilerParams(use_tc_tiling_on_sc=False),
    )


def _bs(shape, const=False):
    if const:
        return pl.BlockSpec(shape, lambda i: (0, 0))
    return pl.BlockSpec(shape, lambda i: (i, 0))


def _make_init():
    n = _N // _NBLK
    return pl.pallas_call(
        _init_body,
        grid=(n,),
        in_specs=[
            _bs((8, _FREQ), True), _bs((_FREQ, _H), True), _bs((1, _H), True),
            _bs((_H, _H), True), _bs((1, _H), True),
            _bs((_NBLK, _H)),
            _bs((_H, _H), True), _bs((1, _H), True),
            _bs((_H, _H), True), _bs((_H, _H), True), _bs((1, _H), True),
        ],
        out_specs=[
            _bs((_NBLK, _H)), _bs((_NBLK, _H)), _bs((_NBLK, _H)),
            _bs((8, _H), True),
        ],
        out_shape=[
            jax.ShapeDtypeStruct((_N, _H), _f32),
            jax.ShapeDtypeStruct((_N, _H), _f32),
            jax.ShapeDtypeStruct((_N, _H), _f32),
            jax.ShapeDtypeStruct((8, _H), _f32),
        ],
    )


def _make_edge():
    n = _EH // _EBLK
    ep = _EBLK // 16
    return pl.pallas_call(
        _edge_body,
        grid=(n,),
        in_specs=[
            _bs((_EBLK, _H)),
            _bs((ep, _H)), _bs((ep, _H)), _bs((ep, 64)),
            _bs((_H, 16), True), _bs((16, _H), True),
            _bs((1, _H), True), _bs((64, _H), True),
            _bs((_H, _H), True), _bs((1, _H), True),
            _bs((_H, _H), True), _bs((1, _H), True), _bs((1, _H), True),
        ],
        out_specs=[_bs((_EBLK, _H)), _bs((ep, _H))],
        out_shape=[
            jax.ShapeDtypeStruct((_EH, _H), _f32),
            jax.ShapeDtypeStruct((_EH // 16, _H), _f32),
        ],
    )


def _make_node():
    n = _N // _NBLK
    return pl.pallas_call(
        _node_body,
        grid=(n,),
        in_specs=[
            _bs((_NBLK, _H)),
            _bs((_NBLK, _H)), _bs((_NBLK, _H)),
            _bs((_NBLK, _H)), _bs((_NBLK, _H)),
            _bs((_NBLK, 8)), _bs((_NBLK, 8)),
            _bs((_NBLK, 8)), _bs((_NBLK, 8)),
            _bs((_NBLK, 8)),
            _bs((8, _H), True),
            _bs((_H, _H), True), _bs((_H, _H), True), _bs((1, _H), True),
            _bs((_H, _H), True), _bs((1, _H), True),
            _bs((_H, _H), True), _bs((_H, _H), True), _bs((1, _H), True),
        ],
        out_specs=[
            _bs((_NBLK, _H)), _bs((_NBLK, 8)),
            _bs((_NBLK, _H)), _bs((_NBLK, _H)),
        ],
        out_shape=[
            jax.ShapeDtypeStruct((_N, _H), _f32),
            jax.ShapeDtypeStruct((_N, 8), _f32),
            jax.ShapeDtypeStruct((_N, _H), _f32),
            jax.ShapeDtypeStruct((_N, _H), _f32),
        ],
    )


def _make_final():
    n = _N // _NBLK
    return pl.pallas_call(
        _final_body,
        grid=(n,),
        in_specs=[
            _bs((_NBLK, _H)),
            _bs((_NBLK, _H)), _bs((_NBLK, _H)),
            _bs((_NBLK, _H)), _bs((_NBLK, _H)),
            _bs((_NBLK, 8)), _bs((_NBLK, 8)),
            _bs((_NBLK, 8)), _bs((_NBLK, 8)),
            _bs((_NBLK, 8)),
            _bs((_H, _H), True), _bs((_H, _H), True), _bs((1, _H), True),
            _bs((_H, _H), True), _bs((1, _H), True),
            _bs((_H, _H), True), _bs((1, _H), True),
        ],
        out_specs=[_bs((_NBLK, _H)), _bs((_NBLK, 8))],
        out_shape=[
            jax.ShapeDtypeStruct((_N, _H), _f32),
            jax.ShapeDtypeStruct((_N, 8), _f32),
        ],
    )


def kernel(h, x, t, edges, edge_attr, params):
    p = params

    half = _FREQ // 2
    freqs = jnp.exp(-np.log(10000.0) * jnp.arange(half, dtype=_f32) / half)
    args = t.astype(_f32)[:, None] * freqs[None]
    tf = jnp.concatenate([jnp.cos(args), jnp.sin(args)], axis=-1)
    tf8 = jnp.broadcast_to(tf, (8, _FREQ))

    xp = jnp.pad(x.astype(_f32), ((0, 0), (0, 5)))
    row1 = edges[0]
    col1 = edges[1]
    ea_p = edge_attr.astype(_f32).reshape(_E // 16, 64)
    zh = jnp.zeros((_N, _H), _f32)
    zx = jnp.zeros((_N, 8), _f32)

    init = _make_init()
    edgek = _make_edge()
    nodek = _make_node()
    finalk = _make_final()
    gath = _make_sc_gather()
    scat = _make_sc_scatter()

    we1 = p['We1'].astype(_f32)
    wn1 = p['Wn1'].astype(_f32)

    grp = np.arange(128) // 8
    smat = jnp.asarray(grp[:, None] == np.arange(16)[None, :], _f32)
    rmat = smat.T

    hh, prow, pcol, temb = init(
        tf8, p['Wt1'], p['bt1'].reshape(1, _H), p['Wt2'], p['bt2'].reshape(1, _H),
        h.astype(_f32), p['W_emb'], p['b_emb'].reshape(1, _H),
        we1[0, :_H], we1[0, _H:2 * _H], p['be1'][0].reshape(1, _H))

    rows = [row1[:_EH], row1[_EH:]]
    cols = [col1[:_EH], col1[_EH:]]
    eas = [ea_p[:_EH // 16], ea_p[_EH // 16:]]

    coord = xp
    h_out = None
    for i in range(_NLAYERS):
        w1r = we1[i, 2 * _H:2 * _H + 1]
        w64 = jnp.tile(we1[i, 2 * _H + 1:], (16, 1))
        phs = []
        pxs = []
        for half in range(2):
            grow, xr, xc = gath(prow, pcol, coord, rows[half], cols[half])
            xrp = xr.reshape(_EH // 16, _H)
            xcp = xc.reshape(_EH // 16, _H)
            ef, trp = edgek(grow, xrp, xcp, eas[half], smat, rmat,
                            w1r, w64,
                            p['We2'][i], p['be2'][i].reshape(1, _H),
                            p['Wc1'][i], p['bc1'][i].reshape(1, _H),
                            p['Wc2'][i].reshape(1, _H))
            tr = trp.reshape(_EH, 8)
            ph, px = scat(ef, tr, rows[half], zh, zx)
            phs += [ph[0], ph[1]]
            pxs += [px[0], px[1]]
        if i < _NLAYERS - 1:
            hh, coord, prow, pcol = nodek(
                hh, phs[0], phs[1], phs[2], phs[3],
                pxs[0], pxs[1], pxs[2], pxs[3], coord, temb,
                wn1[i, :_H], wn1[i, _H:], p['bn1'][i].reshape(1, _H),
                p['Wn2'][i], p['bn2'][i].reshape(1, _H),
                we1[i + 1, :_H], we1[i + 1, _H:2 * _H],
                p['be1'][i + 1].reshape(1, _H))
        else:
            h_out, coord = finalk(
                hh, phs[0], phs[1], phs[2], phs[3],
                pxs[0], pxs[1], pxs[2], pxs[3], coord,
                wn1[i, :_H], wn1[i, _H:], p['bn1'][i].reshape(1, _H),
                p['Wn2'][i], p['bn2'][i].reshape(1, _H),
                p['W_out'], p['b_out'].reshape(1, _H))

    return h_out, coord[:, :3]


# speedup vs baseline: 1.0023x; 1.0023x over previous
import functools
import numpy as np
import jax
import jax.numpy as jnp
from jax import lax
from jax.experimental import pallas as pl
from jax.experimental.pallas import tpu as pltpu
from jax.experimental.pallas import tpu_sc as plsc

_N = 10000
_E = 320000
_H = 128
_NLAYERS = 4
_FREQ = 256

_WORKERS = 32
_EH = _E // 2
_HB = _EH // 128
_BPW = _HB // _WORKERS
_EXTRA = _HB - _BPW * _WORKERS

_NBLK = 1000
_EBLK = 6400

_f32 = jnp.float32
_bf16 = jnp.bfloat16


def _silu(v):
    return v * jax.nn.sigmoid(v)


def _init_body(tf_ref, wt1, bt1, wt2, bt2, h_ref, wemb, bemb, we1a, we1b, be1,
               hh_o, prow_o, pcol_o, temb_o):
    te = _silu(tf_ref[...] @ wt1[...] + bt1[...]) @ wt2[...] + bt2[...]
    temb_o[...] = te
    hh = h_ref[...] @ wemb[...] + bemb[...] + te[0:1, :]
    hh_o[...] = hh
    prow_o[...] = hh @ we1a[...] + be1[...]
    pcol_o[...] = hh @ we1b[...]


def _edge_body(grow, xrp, xcp, eap, smat, rmat, w1r, w64, we2, be2,
               wc1, bc1, wc2, ef_o, tr_o):
    ep = _EBLK // 16
    dp = xrp[...] - xcp[...]
    radial_p = (dp * dp) @ smat[...]
    inv_p = 1.0 / (jnp.sqrt(radial_p + 1e-8) + 1.0)
    mask = (lax.broadcasted_iota(jnp.int32, (_EBLK, 16), 0) % 16
            == lax.broadcasted_iota(jnp.int32, (_EBLK, 16), 1)).astype(_f32)
    rad_x = jax.lax.broadcast_in_dim(radial_p, (ep, 16, 16), (0, 2))
    radial = jnp.sum(rad_x.reshape(_EBLK, 16) * mask, axis=1, keepdims=True)
    ea_x = jax.lax.broadcast_in_dim(eap[...], (ep, 16, 64), (0, 2))
    mask4 = (lax.broadcasted_iota(jnp.int32, (_EBLK, 64), 0) % 16
             == lax.broadcasted_iota(jnp.int32, (_EBLK, 64), 1) // 4).astype(_f32)
    ea_term = (ea_x.reshape(_EBLK, 64) * mask4) @ w64[...]
    m = _silu(grow[...] + radial * w1r[...] + ea_term)
    ef = _silu(jnp.dot(m.astype(_bf16), we2[...].astype(_bf16),
                       preferred_element_type=_f32) + be2[...])
    cm = _silu(jnp.dot(ef.astype(_bf16), wc1[...].astype(_bf16),
                       preferred_element_type=_f32) + bc1[...])
    s = jnp.sum(cm * wc2[...], axis=1, keepdims=True)
    s_p = jnp.sum((s * mask).reshape(ep, 16, 16), axis=1)
    ef_o[...] = ef
    tr_o[...] = dp * ((inv_p * s_p) @ rmat[...])


def _node_body(hh_ref, ph0, ph1, ph2, ph3, px0, px1, px2, px3,
               coord_ref, temb_ref,
               wn1a, wn1b, bn1, wn2, bn2, we1a, we1b, be1,
               hh_o, coord_o, prow_o, pcol_o):
    hh = hh_ref[...]
    aggh = (ph0[...] + ph1[...]) + (ph2[...] + ph3[...])
    o = _silu(hh @ wn1a[...] + aggh @ wn1b[...] + bn1[...]) @ wn2[...] + bn2[...]
    hhn = hh + o + temb_ref[0:1, :]
    hh_o[...] = hhn
    coord_o[...] = coord_ref[...] + (px0[...] + px1[...]) + (px2[...] + px3[...])
    prow_o[...] = hhn @ we1a[...] + be1[...]
    pcol_o[...] = hhn @ we1b[...]


def _final_body(hh_ref, ph0, ph1, ph2, ph3, px0, px1, px2, px3, coord_ref,
                wn1a, wn1b, bn1, wn2, bn2, wout, bout,
                hout_o, coord_o):
    hh = hh_ref[...]
    aggh = (ph0[...] + ph1[...]) + (ph2[...] + ph3[...])
    o = _silu(hh @ wn1a[...] + aggh @ wn1b[...] + bn1[...]) @ wn2[...] + bn2[...]
    hhn = hh + o
    hout_o[...] = hhn @ wout[...] + bout[...]
    coord_o[...] = coord_ref[...] + (px0[...] + px1[...]) + (px2[...] + px3[...])


_MESH = plsc.VectorSubcoreMesh(core_axis_name="c", subcore_axis_name="s")


def _gather_body(prow, pcol, xp, row1, col1,
                 grow_o, xr_o, xc_o,
                 idx_v, g_v, x_v, sems, sh_x):
    cid = lax.axis_index("c")
    sid = lax.axis_index("s")
    wid = sid * 2 + cid

    @pl.when(sid == 0)
    def _():
        pltpu.sync_copy(xp, sh_x)
    plsc.subcore_barrier()

    def start(b, s):
        base = b * 128
        pltpu.sync_copy(row1.at[pl.ds(base, 128)], idx_v[2 * s])
        pltpu.sync_copy(col1.at[pl.ds(base, 128)], idx_v[2 * s + 1])
        pltpu.async_copy(prow.at[idx_v[2 * s]], g_v[2 * s], sems[4 * s])
        pltpu.async_copy(pcol.at[idx_v[2 * s + 1]], g_v[2 * s + 1], sems[4 * s + 1])
        pltpu.async_copy(sh_x.at[idx_v[2 * s]], x_v[2 * s], sems[4 * s + 2])
        pltpu.async_copy(sh_x.at[idx_v[2 * s + 1]], x_v[2 * s + 1], sems[4 * s + 3])

    def finish(b, s):
        base = b * 128
        pltpu.make_async_copy(prow.at[idx_v[2 * s]], g_v[2 * s], sems[4 * s]).wait()
        pltpu.make_async_copy(pcol.at[idx_v[2 * s + 1]], g_v[2 * s + 1], sems[4 * s + 1]).wait()
        pltpu.make_async_copy(sh_x.at[idx_v[2 * s]], x_v[2 * s], sems[4 * s + 2]).wait()
        pltpu.make_async_copy(sh_x.at[idx_v[2 * s + 1]], x_v[2 * s + 1], sems[4 * s + 3]).wait()
        ga = g_v[2 * s]
        gb = g_v[2 * s + 1]

        def addrow(r, carry):
            for c in range(8):
                ga[r, pl.ds(c * 16, 16)] = (ga[r, pl.ds(c * 16, 16)]
                                            + gb[r, pl.ds(c * 16, 16)])
            return carry
        lax.fori_loop(0, 128, addrow, 0)
        pltpu.sync_copy(ga, grow_o.at[pl.ds(base, 128)])
        pltpu.sync_copy(x_v[2 * s], xr_o.at[pl.ds(base, 128)])
        pltpu.sync_copy(x_v[2 * s + 1], xc_o.at[pl.ds(base, 128)])

    base_b = wid * _BPW
    start(base_b, 0)

    def loop(jj, carry):
        b = base_b + 2 * jj
        start(b + 1, 1)
        finish(b, 0)
        start(b + 2, 0)
        finish(b + 1, 1)
        return carry
    lax.fori_loop(0, (_BPW - 1) // 2, loop, 0)
    finish(base_b + _BPW - 1, 0)

    @pl.when(wid < _EXTRA)
    def _():
        start(_WORKERS * _BPW + wid, 1)
        finish(_WORKERS * _BPW + wid, 1)


def _scatter_body(ef, tr, row1, zh, zx, ph_o, px_o,
                  idxb, ef_v, tr_v, sems, sh, sx):
    cid = lax.axis_index("c")
    sid = lax.axis_index("s")
    wid = sid * 2 + cid

    @pl.when(sid == 0)
    def _():
        pltpu.sync_copy(zh, sh)
        pltpu.sync_copy(zx, sx)
    plsc.subcore_barrier()

    def start(b, s):
        base = b * 128
        pltpu.async_copy(row1.at[pl.ds(base, 128)], idxb[s], sems[3 * s])
        pltpu.async_copy(ef.at[pl.ds(base, 128)], ef_v[s], sems[3 * s + 1])
        pltpu.async_copy(tr.at[pl.ds(base, 128)], tr_v[s], sems[3 * s + 2])

    def finish(b, s):
        base = b * 128
        pltpu.make_async_copy(row1.at[pl.ds(base, 128)], idxb[s], sems[3 * s]).wait()
        pltpu.make_async_copy(ef.at[pl.ds(base, 128)], ef_v[s], sems[3 * s + 1]).wait()
        pltpu.make_async_copy(tr.at[pl.ds(base, 128)], tr_v[s], sems[3 * s + 2]).wait()
        pltpu.sync_copy(ef_v[s], sh.at[idxb[s]], add=True)
        pltpu.sync_copy(tr_v[s], sx.at[idxb[s]], add=True)

    base_b = wid * _BPW
    start(base_b, 0)

    def loop(jj, carry):
        b = base_b + 2 * jj
        start(b + 1, 1)
        finish(b, 0)
        start(b + 2, 0)
        finish(b + 1, 1)
        return carry
    lax.fori_loop(0, (_BPW - 1) // 2, loop, 0)
    finish(base_b + _BPW - 1, 0)

    @pl.when(wid < _EXTRA)
    def _():
        start(_WORKERS * _BPW + wid, 1)
        finish(_WORKERS * _BPW + wid, 1)

    plsc.subcore_barrier()

    @pl.when(sid == 0)
    def _():
        pltpu.sync_copy(sh, ph_o.at[cid])
        pltpu.sync_copy(sx, px_o.at[cid])


def _make_sc_gather():
    return pl.kernel(
        _gather_body,
        out_type=(
            jax.ShapeDtypeStruct((_EH, _H), _f32),
            jax.ShapeDtypeStruct((_EH, 8), _f32),
            jax.ShapeDtypeStruct((_EH, 8), _f32),
        ),
        mesh=_MESH,
        scratch_types=[
            [pltpu.VMEM((128,), jnp.int32) for _ in range(4)],
            [pltpu.VMEM((128, _H), _f32) for _ in range(4)],
            [pltpu.VMEM((128, 8), _f32) for _ in range(4)],
            [pltpu.SemaphoreType.DMA for _ in range(8)],
            pltpu.VMEM_SHARED((_N, 8), _f32),
        ],
        compiler_params=pltpu.CompilerParams(use_tc_tiling_on_sc=False),
    )


def _make_sc_scatter():
    return pl.kernel(
        _scatter_body,
        out_type=(
            jax.ShapeDtypeStruct((2, _N, _H), _f32),
            jax.ShapeDtypeStruct((2, _N, 8), _f32),
        ),
        mesh=_MESH,
        scratch_types=[
            [pltpu.VMEM((128,), jnp.int32) for _ in range(2)],
            [pltpu.VMEM((128, _H), _f32) for _ in range(2)],
            [pltpu.VMEM((128, 8), _f32) for _ in range(2)],
            [pltpu.SemaphoreType.DMA for _ in range(6)],
            pltpu.VMEM_SHARED((_N, _H), _f32),
            pltpu.VMEM_SHARED((_N, 8), _f32),
        ],
        compiler_params=pltpu.CompilerParams(use_tc_tiling_on_sc=False),
    )


def _bs(shape, const=False):
    if const:
        return pl.BlockSpec(shape, lambda i: (0, 0))
    return pl.BlockSpec(shape, lambda i: (i, 0))


def _make_init():
    n = _N // _NBLK
    return pl.pallas_call(
        _init_body,
        grid=(n,),
        in_specs=[
            _bs((8, _FREQ), True), _bs((_FREQ, _H), True), _bs((1, _H), True),
            _bs((_H, _H), True), _bs((1, _H), True),
            _bs((_NBLK, _H)),
            _bs((_H, _H), True), _bs((1, _H), True),
            _bs((_H, _H), True), _bs((_H, _H), True), _bs((1, _H), True),
        ],
        out_specs=[
            _bs((_NBLK, _H)), _bs((_NBLK, _H)), _bs((_NBLK, _H)),
            _bs((8, _H), True),
        ],
        out_shape=[
            jax.ShapeDtypeStruct((_N, _H), _f32),
            jax.ShapeDtypeStruct((_N, _H), _f32),
            jax.ShapeDtypeStruct((_N, _H), _f32),
            jax.ShapeDtypeStruct((8, _H), _f32),
        ],
    )


def _make_edge():
    n = _EH // _EBLK
    ep = _EBLK // 16
    return pl.pallas_call(
        _edge_body,
        grid=(n,),
        in_specs=[
            _bs((_EBLK, _H)),
            _bs((ep, _H)), _bs((ep, _H)), _bs((ep, 64)),
            _bs((_H, 16), True), _bs((16, _H), True),
            _bs((1, _H), True), _bs((64, _H), True),
            _bs((_H, _H), True), _bs((1, _H), True),
            _bs((_H, _H), True), _bs((1, _H), True), _bs((1, _H), True),
        ],
        out_specs=[_bs((_EBLK, _H)), _bs((ep, _H))],
        out_shape=[
            jax.ShapeDtypeStruct((_EH, _H), _f32),
            jax.ShapeDtypeStruct((_EH // 16, _H), _f32),
        ],
    )


def _make_node():
    n = _N // _NBLK
    return pl.pallas_call(
        _node_body,
        grid=(n,),
        in_specs=[
            _bs((_NBLK, _H)),
            _bs((_NBLK, _H)), _bs((_NBLK, _H)),
            _bs((_NBLK, _H)), _bs((_NBLK, _H)),
            _bs((_NBLK, 8)), _bs((_NBLK, 8)),
            _bs((_NBLK, 8)), _bs((_NBLK, 8)),
            _bs((_NBLK, 8)),
            _bs((8, _H), True),
            _bs((_H, _H), True), _bs((_H, _H), True), _bs((1, _H), True),
            _bs((_H, _H), True), _bs((1, _H), True),
            _bs((_H, _H), True), _bs((_H, _H), True), _bs((1, _H), True),
        ],
        out_specs=[
            _bs((_NBLK, _H)), _bs((_NBLK, 8)),
            _bs((_NBLK, _H)), _bs((_NBLK, _H)),
        ],
        out_shape=[
            jax.ShapeDtypeStruct((_N, _H), _f32),
            jax.ShapeDtypeStruct((_N, 8), _f32),
            jax.ShapeDtypeStruct((_N, _H), _f32),
            jax.ShapeDtypeStruct((_N, _H), _f32),
        ],
    )


def _make_final():
    n = _N // _NBLK
    return pl.pallas_call(
        _final_body,
        grid=(n,),
        in_specs=[
            _bs((_NBLK, _H)),
            _bs((_NBLK, _H)), _bs((_NBLK, _H)),
            _bs((_NBLK, _H)), _bs((_NBLK, _H)),
            _bs((_NBLK, 8)), _bs((_NBLK, 8)),
            _bs((_NBLK, 8)), _bs((_NBLK, 8)),
            _bs((_NBLK, 8)),
            _bs((_H, _H), True), _bs((_H, _H), True), _bs((1, _H), True),
            _bs((_H, _H), True), _bs((1, _H), True),
            _bs((_H, _H), True), _bs((1, _H), True),
        ],
        out_specs=[_bs((_NBLK, _H)), _bs((_NBLK, 8))],
        out_shape=[
            jax.ShapeDtypeStruct((_N, _H), _f32),
            jax.ShapeDtypeStruct((_N, 8), _f32),
        ],
    )


def kernel(h, x, t, edges, edge_attr, params):
    p = params

    half = _FREQ // 2
    freqs = jnp.exp(-np.log(10000.0) * jnp.arange(half, dtype=_f32) / half)
    args = t.astype(_f32)[:, None] * freqs[None]
    tf = jnp.concatenate([jnp.cos(args), jnp.sin(args)], axis=-1)
    tf8 = jnp.broadcast_to(tf, (8, _FREQ))

    xp = jnp.pad(x.astype(_f32), ((0, 0), (0, 5)))
    row1 = edges[0]
    col1 = edges[1]
    ea_p = edge_attr.astype(_f32).reshape(_E // 16, 64)
    zh = jnp.zeros((_N, _H), _f32)
    zx = jnp.zeros((_N, 8), _f32)

    init = _make_init()
    edgek = _make_edge()
    nodek = _make_node()
    finalk = _make_final()
    gath = _make_sc_gather()
    scat = _make_sc_scatter()

    we1 = p['We1'].astype(_f32)
    wn1 = p['Wn1'].astype(_f32)

    grp = np.arange(128) // 8
    smat = jnp.asarray(grp[:, None] == np.arange(16)[None, :], _f32)
    rmat = smat.T

    hh, prow, pcol, temb = init(
        tf8, p['Wt1'], p['bt1'].reshape(1, _H), p['Wt2'], p['bt2'].reshape(1, _H),
        h.astype(_f32), p['W_emb'], p['b_emb'].reshape(1, _H),
        we1[0, :_H], we1[0, _H:2 * _H], p['be1'][0].reshape(1, _H))

    rows = [row1[:_EH], row1[_EH:]]
    cols = [col1[:_EH], col1[_EH:]]
    eas = [ea_p[:_EH // 16], ea_p[_EH // 16:]]

    coord = xp
    h_out = None
    for i in range(_NLAYERS):
        w1r = we1[i, 2 * _H:2 * _H + 1]
        w64 = jnp.tile(we1[i, 2 * _H + 1:], (16, 1))
        phs = []
        pxs = []
        for half in range(2):
            grow, xr, xc = gath(prow, pcol, coord, rows[half], cols[half])
            xrp = xr.reshape(_EH // 16, _H)
            xcp = xc.reshape(_EH // 16, _H)
            ef, trp = edgek(grow, xrp, xcp, eas[half], smat, rmat,
                            w1r, w64,
                            p['We2'][i], p['be2'][i].reshape(1, _H),
                            p['Wc1'][i], p['bc1'][i].reshape(1, _H),
                            p['Wc2'][i].reshape(1, _H))
            tr = trp.reshape(_EH, 8)
            ph, px = scat(ef, tr, rows[half], zh, zx)
            phs += [ph[0], ph[1]]
            pxs += [px[0], px[1]]
        if i < _NLAYERS - 1:
            hh, coord, prow, pcol = nodek(
                hh, phs[0], phs[1], phs[2], phs[3],
                pxs[0], pxs[1], pxs[2], pxs[3], coord, temb,
                wn1[i, :_H], wn1[i, _H:], p['bn1'][i].reshape(1, _H),
                p['Wn2'][i], p['bn2'][i].reshape(1, _H),
                we1[i + 1, :_H], we1[i + 1, _H:2 * _H],
                p['be1'][i + 1].reshape(1, _H))
        else:
            h_out, coord = finalk(
                hh, phs[0], phs[1], phs[2], phs[3],
                pxs[0], pxs[1], pxs[2], pxs[3], coord,
                wn1[i, :_H], wn1[i, _H:], p['bn1'][i].reshape(1, _H),
                p['Wn2'][i], p['bn2'][i].reshape(1, _H),
                p['W_out'], p['b_out'].reshape(1, _H))

    return h_out, coord[:, :3]


# speedup vs baseline: 1.0068x; 1.0045x over previous
import functools
import numpy as np
import jax
import jax.numpy as jnp
from jax import lax
from jax.experimental import pallas as pl
from jax.experimental.pallas import tpu as pltpu
from jax.experimental.pallas import tpu_sc as plsc

_N = 10000
_E = 320000
_H = 128
_NLAYERS = 4
_FREQ = 256

_WORKERS = 32
_EH = _E // 2
_HB = _EH // 128
_BPW = _HB // _WORKERS
_EXTRA = _HB - _BPW * _WORKERS

_NBLK = 1000
_EBLK = 6400

_f32 = jnp.float32


def _silu(v):
    return v * jax.nn.sigmoid(v)


def _init_body(tf_ref, wt1, bt1, wt2, bt2, h_ref, wemb, bemb, we1a, we1b, be1,
               hh_o, prow_o, pcol_o, temb_o):
    te = _silu(tf_ref[...] @ wt1[...] + bt1[...]) @ wt2[...] + bt2[...]
    temb_o[...] = te
    hh = h_ref[...] @ wemb[...] + bemb[...] + te[0:1, :]
    hh_o[...] = hh
    prow_o[...] = hh @ we1a[...] + be1[...]
    pcol_o[...] = hh @ we1b[...]


def _edge_body(grow, xrp, xcp, eap, smat, rmat, w1r, w64, we2, be2,
               wc1, bc1, wc2, ef_o, tr_o):
    ep = _EBLK // 16
    dp = xrp[...] - xcp[...]
    radial_p = (dp * dp) @ smat[...]
    inv_p = 1.0 / (jnp.sqrt(radial_p + 1e-8) + 1.0)
    mask = (lax.broadcasted_iota(jnp.int32, (_EBLK, 16), 0) % 16
            == lax.broadcasted_iota(jnp.int32, (_EBLK, 16), 1)).astype(_f32)
    rad_x = jax.lax.broadcast_in_dim(radial_p, (ep, 16, 16), (0, 2))
    radial = jnp.sum(rad_x.reshape(_EBLK, 16) * mask, axis=1, keepdims=True)
    ea_x = jax.lax.broadcast_in_dim(eap[...], (ep, 16, 64), (0, 2))
    mask4 = (lax.broadcasted_iota(jnp.int32, (_EBLK, 64), 0) % 16
             == lax.broadcasted_iota(jnp.int32, (_EBLK, 64), 1) // 4).astype(_f32)
    ea_term = (ea_x.reshape(_EBLK, 64) * mask4) @ w64[...]
    m = _silu(grow[...] + radial * w1r[...] + ea_term)
    ef = _silu(m @ we2[...] + be2[...])
    cm = _silu(ef @ wc1[...] + bc1[...])
    s = jnp.sum(cm * wc2[...], axis=1, keepdims=True)
    s_p = jnp.sum((s * mask).reshape(ep, 16, 16), axis=1)
    ef_o[...] = ef
    tr_o[...] = dp * ((inv_p * s_p) @ rmat[...])


def _node_body(hh_ref, ph0, ph1, ph2, ph3, px0, px1, px2, px3,
               coord_ref, temb_ref,
               wn1a, wn1b, bn1, wn2, bn2, we1a, we1b, be1,
               hh_o, coord_o, prow_o, pcol_o):
    hh = hh_ref[...]
    aggh = (ph0[...] + ph1[...]) + (ph2[...] + ph3[...])
    o = _silu(hh @ wn1a[...] + aggh @ wn1b[...] + bn1[...]) @ wn2[...] + bn2[...]
    hhn = hh + o + temb_ref[0:1, :]
    hh_o[...] = hhn
    coord_o[...] = coord_ref[...] + (px0[...] + px1[...]) + (px2[...] + px3[...])
    prow_o[...] = hhn @ we1a[...] + be1[...]
    pcol_o[...] = hhn @ we1b[...]


def _final_body(hh_ref, ph0, ph1, ph2, ph3, px0, px1, px2, px3, coord_ref,
                wn1a, wn1b, bn1, wn2, bn2, wout, bout,
                hout_o, coord_o):
    hh = hh_ref[...]
    aggh = (ph0[...] + ph1[...]) + (ph2[...] + ph3[...])
    o = _silu(hh @ wn1a[...] + aggh @ wn1b[...] + bn1[...]) @ wn2[...] + bn2[...]
    hhn = hh + o
    hout_o[...] = hhn @ wout[...] + bout[...]
    coord_o[...] = coord_ref[...] + (px0[...] + px1[...]) + (px2[...] + px3[...])


_MESH = plsc.VectorSubcoreMesh(core_axis_name="c", subcore_axis_name="s")


def _gather_body(prow, pcol, xp, row1, col1,
                 grow_o, xr_o, xc_o,
                 idx_v, g_v, x_v, sems, sh_x):
    cid = lax.axis_index("c")
    sid = lax.axis_index("s")
    wid = sid * 2 + cid

    @pl.when(sid == 0)
    def _():
        pltpu.sync_copy(xp, sh_x)
    plsc.subcore_barrier()

    def start(b, s):
        base = b * 128
        pltpu.sync_copy(row1.at[pl.ds(base, 128)], idx_v[2 * s])
        pltpu.sync_copy(col1.at[pl.ds(base, 128)], idx_v[2 * s + 1])
        pltpu.async_copy(prow.at[idx_v[2 * s]], g_v[2 * s], sems[4 * s])
        pltpu.async_copy(pcol.at[idx_v[2 * s + 1]], g_v[2 * s + 1], sems[4 * s + 1])
        pltpu.async_copy(sh_x.at[idx_v[2 * s]], x_v[2 * s], sems[4 * s + 2])
        pltpu.async_copy(sh_x.at[idx_v[2 * s + 1]], x_v[2 * s + 1], sems[4 * s + 3])

    def finish(b, s):
        base = b * 128
        pltpu.make_async_copy(prow.at[idx_v[2 * s]], g_v[2 * s], sems[4 * s]).wait()
        pltpu.make_async_copy(pcol.at[idx_v[2 * s + 1]], g_v[2 * s + 1], sems[4 * s + 1]).wait()
        pltpu.make_async_copy(sh_x.at[idx_v[2 * s]], x_v[2 * s], sems[4 * s + 2]).wait()
        pltpu.make_async_copy(sh_x.at[idx_v[2 * s + 1]], x_v[2 * s + 1], sems[4 * s + 3]).wait()
        ga = g_v[2 * s]
        gb = g_v[2 * s + 1]

        def addrow(r, carry):
            for c in range(8):
                ga[r, pl.ds(c * 16, 16)] = (ga[r, pl.ds(c * 16, 16)]
                                            + gb[r, pl.ds(c * 16, 16)])
            return carry
        lax.fori_loop(0, 128, addrow, 0)
        pltpu.sync_copy(ga, grow_o.at[pl.ds(base, 128)])
        pltpu.sync_copy(x_v[2 * s], xr_o.at[pl.ds(base, 128)])
        pltpu.sync_copy(x_v[2 * s + 1], xc_o.at[pl.ds(base, 128)])

    base_b = wid * _BPW
    start(base_b, 0)

    def loop(jj, carry):
        b = base_b + 2 * jj
        start(b + 1, 1)
        finish(b, 0)
        start(b + 2, 0)
        finish(b + 1, 1)
        return carry
    lax.fori_loop(0, (_BPW - 1) // 2, loop, 0)
    finish(base_b + _BPW - 1, 0)

    @pl.when(wid < _EXTRA)
    def _():
        start(_WORKERS * _BPW + wid, 1)
        finish(_WORKERS * _BPW + wid, 1)


def _scatter_body(ef, tr, row1, zh, zx, ph_o, px_o,
                  idxb, ef_v, tr_v, sems, sh, sx):
    cid = lax.axis_index("c")
    sid = lax.axis_index("s")
    wid = sid * 2 + cid

    @pl.when(sid == 0)
    def _():
        pltpu.sync_copy(zh, sh)
        pltpu.sync_copy(zx, sx)
    plsc.subcore_barrier()

    def start(b, s):
        base = b * 128
        pltpu.async_copy(row1.at[pl.ds(base, 128)], idxb[s], sems[3 * s])
        pltpu.async_copy(ef.at[pl.ds(base, 128)], ef_v[s], sems[3 * s + 1])
        pltpu.async_copy(tr.at[pl.ds(base, 128)], tr_v[s], sems[3 * s + 2])

    def finish(b, s):
        base = b * 128
        pltpu.make_async_copy(row1.at[pl.ds(base, 128)], idxb[s], sems[3 * s]).wait()
        pltpu.make_async_copy(ef.at[pl.ds(base, 128)], ef_v[s], sems[3 * s + 1]).wait()
        pltpu.make_async_copy(tr.at[pl.ds(base, 128)], tr_v[s], sems[3 * s + 2]).wait()
        pltpu.sync_copy(ef_v[s], sh.at[idxb[s]], add=True)
        pltpu.sync_copy(tr_v[s], sx.at[idxb[s]], add=True)

    base_b = wid * _BPW
    start(base_b, 0)

    def loop(jj, carry):
        b = base_b + 2 * jj
        start(b + 1, 1)
        finish(b, 0)
        start(b + 2, 0)
        finish(b + 1, 1)
        return carry
    lax.fori_loop(0, (_BPW - 1) // 2, loop, 0)
    finish(base_b + _BPW - 1, 0)

    @pl.when(wid < _EXTRA)
    def _():
        start(_WORKERS * _BPW + wid, 1)
        finish(_WORKERS * _BPW + wid, 1)

    plsc.subcore_barrier()

    @pl.when(sid == 0)
    def _():
        pltpu.sync_copy(sh, ph_o.at[cid])
        pltpu.sync_copy(sx, px_o.at[cid])


def _make_sc_gather():
    return pl.kernel(
        _gather_body,
        out_type=(
            jax.ShapeDtypeStruct((_EH, _H), _f32),
            jax.ShapeDtypeStruct((_EH, 8), _f32),
            jax.ShapeDtypeStruct((_EH, 8), _f32),
        ),
        mesh=_MESH,
        scratch_types=[
            [pltpu.VMEM((128,), jnp.int32) for _ in range(4)],
            [pltpu.VMEM((128, _H), _f32) for _ in range(4)],
            [pltpu.VMEM((128, 8), _f32) for _ in range(4)],
            [pltpu.SemaphoreType.DMA for _ in range(8)],
            pltpu.VMEM_SHARED((_N, 8), _f32),
        ],
        compiler_params=pltpu.CompilerParams(use_tc_tiling_on_sc=False),
    )


def _make_sc_scatter():
    return pl.kernel(
        _scatter_body,
        out_type=(
            jax.ShapeDtypeStruct((2, _N, _H), _f32),
            jax.ShapeDtypeStruct((2, _N, 8), _f32),
        ),
        mesh=_MESH,
        scratch_types=[
            [pltpu.VMEM((128,), jnp.int32) for _ in range(2)],
            [pltpu.VMEM((128, _H), _f32) for _ in range(2)],
            [pltpu.VMEM((128, 8), _f32) for _ in range(2)],
            [pltpu.SemaphoreType.DMA for _ in range(6)],
            pltpu.VMEM_SHARED((_N, _H), _f32),
            pltpu.VMEM_SHARED((_N, 8), _f32),
        ],
        compiler_params=pltpu.CompilerParams(use_tc_tiling_on_sc=False),
    )


def _bs(shape, const=False):
    if const:
        return pl.BlockSpec(shape, lambda i: (0, 0))
    return pl.BlockSpec(shape, lambda i: (i, 0))


def _make_init():
    n = _N // _NBLK
    return pl.pallas_call(
        _init_body,
        grid=(n,),
        in_specs=[
            _bs((8, _FREQ), True), _bs((_FREQ, _H), True), _bs((1, _H), True),
            _bs((_H, _H), True), _bs((1, _H), True),
            _bs((_NBLK, _H)),
            _bs((_H, _H), True), _bs((1, _H), True),
            _bs((_H, _H), True), _bs((_H, _H), True), _bs((1, _H), True),
        ],
        out_specs=[
            _bs((_NBLK, _H)), _bs((_NBLK, _H)), _bs((_NBLK, _H)),
            _bs((8, _H), True),
        ],
        out_shape=[
            jax.ShapeDtypeStruct((_N, _H), _f32),
            jax.ShapeDtypeStruct((_N, _H), _f32),
            jax.ShapeDtypeStruct((_N, _H), _f32),
            jax.ShapeDtypeStruct((8, _H), _f32),
        ],
    )


def _make_edge():
    n = _EH // _EBLK
    ep = _EBLK // 16
    return pl.pallas_call(
        _edge_body,
        grid=(n,),
        in_specs=[
            _bs((_EBLK, _H)),
            _bs((ep, _H)), _bs((ep, _H)), _bs((ep, 64)),
            _bs((_H, 16), True), _bs((16, _H), True),
            _bs((1, _H), True), _bs((64, _H), True),
            _bs((_H, _H), True), _bs((1, _H), True),
            _bs((_H, _H), True), _bs((1, _H), True), _bs((1, _H), True),
        ],
        out_specs=[_bs((_EBLK, _H)), _bs((ep, _H))],
        out_shape=[
            jax.ShapeDtypeStruct((_EH, _H), _f32),
            jax.ShapeDtypeStruct((_EH // 16, _H), _f32),
        ],
    )


def _make_node():
    n = _N // _NBLK
    return pl.pallas_call(
        _node_body,
        grid=(n,),
        in_specs=[
            _bs((_NBLK, _H)),
            _bs((_NBLK, _H)), _bs((_NBLK, _H)),
            _bs((_NBLK, _H)), _bs((_NBLK, _H)),
            _bs((_NBLK, 8)), _bs((_NBLK, 8)),
            _bs((_NBLK, 8)), _bs((_NBLK, 8)),
            _bs((_NBLK, 8)),
            _bs((8, _H), True),
            _bs((_H, _H), True), _bs((_H, _H), True), _bs((1, _H), True),
            _bs((_H, _H), True), _bs((1, _H), True),
            _bs((_H, _H), True), _bs((_H, _H), True), _bs((1, _H), True),
        ],
        out_specs=[
            _bs((_NBLK, _H)), _bs((_NBLK, 8)),
            _bs((_NBLK, _H)), _bs((_NBLK, _H)),
        ],
        out_shape=[
            jax.ShapeDtypeStruct((_N, _H), _f32),
            jax.ShapeDtypeStruct((_N, 8), _f32),
            jax.ShapeDtypeStruct((_N, _H), _f32),
            jax.ShapeDtypeStruct((_N, _H), _f32),
        ],
    )


def _make_final():
    n = _N // _NBLK
    return pl.pallas_call(
        _final_body,
        grid=(n,),
        in_specs=[
            _bs((_NBLK, _H)),
            _bs((_NBLK, _H)), _bs((_NBLK, _H)),
            _bs((_NBLK, _H)), _bs((_NBLK, _H)),
            _bs((_NBLK, 8)), _bs((_NBLK, 8)),
            _bs((_NBLK, 8)), _bs((_NBLK, 8)),
            _bs((_NBLK, 8)),
            _bs((_H, _H), True), _bs((_H, _H), True), _bs((1, _H), True),
            _bs((_H, _H), True), _bs((1, _H), True),
            _bs((_H, _H), True), _bs((1, _H), True),
        ],
        out_specs=[_bs((_NBLK, _H)), _bs((_NBLK, 8))],
        out_shape=[
            jax.ShapeDtypeStruct((_N, _H), _f32),
            jax.ShapeDtypeStruct((_N, 8), _f32),
        ],
    )


def kernel(h, x, t, edges, edge_attr, params):
    p = params

    half = _FREQ // 2
    freqs = jnp.exp(-np.log(10000.0) * jnp.arange(half, dtype=_f32) / half)
    args = t.astype(_f32)[:, None] * freqs[None]
    tf = jnp.concatenate([jnp.cos(args), jnp.sin(args)], axis=-1)
    tf8 = jnp.broadcast_to(tf, (8, _FREQ))

    xp = jnp.pad(x.astype(_f32), ((0, 0), (0, 5)))
    row1 = edges[0]
    col1 = edges[1]
    ea_p = edge_attr.astype(_f32).reshape(_E // 16, 64)
    zh = jnp.zeros((_N, _H), _f32)
    zx = jnp.zeros((_N, 8), _f32)

    init = _make_init()
    edgek = _make_edge()
    nodek = _make_node()
    finalk = _make_final()
    gath = _make_sc_gather()
    scat = _make_sc_scatter()

    we1 = p['We1'].astype(_f32)
    wn1 = p['Wn1'].astype(_f32)

    grp = np.arange(128) // 8
    smat = jnp.asarray(grp[:, None] == np.arange(16)[None, :], _f32)
    rmat = smat.T

    hh, prow, pcol, temb = init(
        tf8, p['Wt1'], p['bt1'].reshape(1, _H), p['Wt2'], p['bt2'].reshape(1, _H),
        h.astype(_f32), p['W_emb'], p['b_emb'].reshape(1, _H),
        we1[0, :_H], we1[0, _H:2 * _H], p['be1'][0].reshape(1, _H))

    rows = [row1[:_EH], row1[_EH:]]
    cols = [col1[:_EH], col1[_EH:]]
    eas = [ea_p[:_EH // 16], ea_p[_EH // 16:]]

    coord = xp
    h_out = None
    for i in range(_NLAYERS):
        w1r = we1[i, 2 * _H:2 * _H + 1]
        w64 = jnp.tile(we1[i, 2 * _H + 1:], (16, 1))
        phs = []
        pxs = []
        for half in range(2):
            grow, xr, xc = gath(prow, pcol, coord, rows[half], cols[half])
            xrp = xr.reshape(_EH // 16, _H)
            xcp = xc.reshape(_EH // 16, _H)
            ef, trp = edgek(grow, xrp, xcp, eas[half], smat, rmat,
                            w1r, w64,
                            p['We2'][i], p['be2'][i].reshape(1, _H),
                            p['Wc1'][i], p['bc1'][i].reshape(1, _H),
                            p['Wc2'][i].reshape(1, _H))
            tr = trp.reshape(_EH, 8)
            ph, px = scat(ef, tr, rows[half], zh, zx)
            phs += [ph[0], ph[1]]
            pxs += [px[0], px[1]]
        if i < _NLAYERS - 1:
            hh, coord, prow, pcol = nodek(
                hh, phs[0], phs[1], phs[2], phs[3],
                pxs[0], pxs[1], pxs[2], pxs[3], coord, temb,
                wn1[i, :_H], wn1[i, _H:], p['bn1'][i].reshape(1, _H),
                p['Wn2'][i], p['bn2'][i].reshape(1, _H),
                we1[i + 1, :_H], we1[i + 1, _H:2 * _H],
                p['be1'][i + 1].reshape(1, _H))
        else:
            h_out, coord = finalk(
                hh, phs[0], phs[1], phs[2], phs[3],
                pxs[0], pxs[1], pxs[2], pxs[3], coord,
                wn1[i, :_H], wn1[i, _H:], p['bn1'][i].reshape(1, _H),
                p['Wn2'][i], p['bn2'][i].reshape(1, _H),
                p['W_out'], p['b_out'].reshape(1, _H))

    return h_out, coord[:, :3]
